# Initial kernel scaffold; baseline (speedup 1.0000x reference)
import jax
import jax.numpy as jnp
from jax.experimental import pallas as pl


def kernel(x, edge_index, batch, W_embed, W_gcn, b_gcn, centroids, W_out, b_out):
    raise NotImplementedError("write your pallas kernel here")


# speedup vs baseline: 12.6480x; 12.6480x over previous
import functools

import jax
import jax.numpy as jnp
from jax import lax
from jax.experimental import pallas as pl
from jax.experimental.pallas import tpu as pltpu
from jax.experimental.pallas import tpu_sc as plsc

N = 10000
E = 320000
D = 128
NUM_CENTROID = 100
NUM_CLASS = 10
NUM_GRAPHS = 128

NC = 2
NS = 16
ACC_PAD = 10240
ROWS_PER_TILE = ACC_PAD // NS
EDGES_PER_CORE = E // NC
EDGES_PER_TILE = E // (NC * NS)
ECHUNK = 80
NCHUNK = EDGES_PER_TILE // ECHUNK
DEG_PAD = 10240
DEG_PER_TILE = DEG_PAD // NS

_HIGH = jax.lax.Precision.HIGHEST


def _mesh():
    return plsc.VectorSubcoreMesh(core_axis_name="c", subcore_axis_name="s")


def _deg_body(dst_hbm, deg_hbm, acc_sh, dst_v, ones_v, zbuf):
    c = lax.axis_index("c")
    s = lax.axis_index("s")
    one16 = jnp.full((16,), 1.0, dtype=jnp.float32)
    zero16 = jnp.zeros((16,), dtype=jnp.float32)

    def fill_ones(k, _):
        ones_v[pl.ds(k * 16, 16)] = one16
        return 0

    lax.fori_loop(0, ECHUNK // 16, fill_ones, 0)

    def fill_zero(k, _):
        zbuf[pl.ds(k * 16, 16)] = zero16
        return 0

    lax.fori_loop(0, DEG_PER_TILE // 16, fill_zero, 0)
    pltpu.sync_copy(zbuf, acc_sh.at[pl.ds(s * DEG_PER_TILE, DEG_PER_TILE)])
    plsc.subcore_barrier()

    ebase = (c * NS + s) * EDGES_PER_TILE

    def step(it, _):
        off = ebase + it * ECHUNK
        pltpu.sync_copy(dst_hbm.at[pl.ds(off, ECHUNK)], dst_v)
        pltpu.sync_copy(ones_v, acc_sh.at[dst_v], add=True)
        return 0

    lax.fori_loop(0, NCHUNK, step, 0)
    plsc.subcore_barrier()
    pltpu.sync_copy(
        acc_sh.at[pl.ds(s * DEG_PER_TILE, DEG_PER_TILE)],
        deg_hbm.at[pl.ds(c * DEG_PAD + s * DEG_PER_TILE, DEG_PER_TILE)],
    )


def _sc_degree(dst):
    kern = pl.kernel(
        _deg_body,
        out_type=jax.ShapeDtypeStruct((NC * DEG_PAD,), jnp.float32),
        mesh=_mesh(),
        scratch_types=[
            pltpu.VMEM_SHARED((DEG_PAD,), jnp.float32),
            pltpu.VMEM((ECHUNK,), jnp.int32),
            pltpu.VMEM((ECHUNK,), jnp.float32),
            pltpu.VMEM((DEG_PER_TILE,), jnp.float32),
        ],
    )
    return kern(dst)


def _agg_body(g_hbm, src_hbm, dst_hbm, out_hbm, acc_sh, src_v, dst_v, rows_v,
              zbuf, sem):
    c = lax.axis_index("c")
    s = lax.axis_index("s")
    zero16 = jnp.zeros((16,), dtype=jnp.float32)

    def zrow(r, _):
        for j in range(D // 16):
            zbuf[r, pl.ds(j * 16, 16)] = zero16
        return 0

    lax.fori_loop(0, 128, zrow, 0)
    for rr in range(ROWS_PER_TILE // 128):
        pltpu.sync_copy(zbuf, acc_sh.at[pl.ds(s * ROWS_PER_TILE + rr * 128, 128)])
    plsc.subcore_barrier()

    ebase = (c * NS + s) * EDGES_PER_TILE

    def step(it, _):
        off = ebase + it * ECHUNK
        pltpu.sync_copy(src_hbm.at[pl.ds(off, ECHUNK)], src_v)
        pltpu.sync_copy(dst_hbm.at[pl.ds(off, ECHUNK)], dst_v)
        pltpu.async_copy(g_hbm.at[src_v], rows_v, sem).wait()
        pltpu.sync_copy(rows_v, acc_sh.at[dst_v], add=True)
        return 0

    lax.fori_loop(0, NCHUNK, step, 0)
    plsc.subcore_barrier()
    pltpu.sync_copy(
        acc_sh.at[pl.ds(s * ROWS_PER_TILE, ROWS_PER_TILE)],
        out_hbm.at[pl.ds(c * ACC_PAD + s * ROWS_PER_TILE, ROWS_PER_TILE)],
    )


def _sc_aggregate(g, src, dst):
    kern = pl.kernel(
        _agg_body,
        out_type=jax.ShapeDtypeStruct((NC * ACC_PAD, D), jnp.float32),
        mesh=_mesh(),
        scratch_types=[
            pltpu.VMEM_SHARED((ACC_PAD, D), jnp.float32),
            pltpu.VMEM((ECHUNK,), jnp.int32),
            pltpu.VMEM((ECHUNK,), jnp.int32),
            pltpu.VMEM((ECHUNK, D), jnp.float32),
            pltpu.VMEM((128, D), jnp.float32),
            pltpu.SemaphoreType.DMA,
        ],
    )
    return kern(g, src, dst)


_RB = 1000
_GRID = N // _RB


def _k2_body(x_ref, degp_ref, we_ref, w0_ref, g1_ref, dinv_ref):
    deg = degp_ref[0, :, 0] + degp_ref[1, :, 0] + 1.0
    dv = lax.rsqrt(deg)
    t = lax.dot_general(x_ref[...], we_ref[...], (((1,), (0,)), ((), ())),
                        precision=_HIGH, preferred_element_type=jnp.float32)
    t = lax.dot_general(t, w0_ref[...], (((1,), (0,)), ((), ())),
                        precision=_HIGH, preferred_element_type=jnp.float32)
    g1_ref[...] = dv[:, None] * t
    dinv_ref[...] = dv[:, None]


def _tc_embed_scale(x, degp, W_embed, W0):
    return pl.pallas_call(
        _k2_body,
        grid=(_GRID,),
        in_specs=[
            pl.BlockSpec((_RB, D), lambda i: (i, 0)),
            pl.BlockSpec((NC, _RB, 1), lambda i: (0, i, 0)),
            pl.BlockSpec((D, D), lambda i: (0, 0)),
            pl.BlockSpec((D, D), lambda i: (0, 0)),
        ],
        out_specs=[
            pl.BlockSpec((_RB, D), lambda i: (i, 0)),
            pl.BlockSpec((_RB, 1), lambda i: (i, 0)),
        ],
        out_shape=[
            jax.ShapeDtypeStruct((N, D), jnp.float32),
            jax.ShapeDtypeStruct((N, 1), jnp.float32),
        ],
    )(x, degp, W_embed, W0)


def _k4_body(acc_ref, g1_ref, dinv_ref, w1_ref, b0_ref, g2_ref):
    dv = dinv_ref[...]
    a = acc_ref[0] + acc_ref[1] + g1_ref[...]
    h1 = jnp.maximum(dv * a + b0_ref[...], 0.0)
    t = lax.dot_general(h1, w1_ref[...], (((1,), (0,)), ((), ())),
                        precision=_HIGH, preferred_element_type=jnp.float32)
    g2_ref[...] = dv * t


def _tc_layer2_prep(acc1, g1, dinv, W1, b0):
    return pl.pallas_call(
        _k4_body,
        grid=(_GRID,),
        in_specs=[
            pl.BlockSpec((NC, _RB, D), lambda i: (0, i, 0)),
            pl.BlockSpec((_RB, D), lambda i: (i, 0)),
            pl.BlockSpec((_RB, 1), lambda i: (i, 0)),
            pl.BlockSpec((D, D), lambda i: (0, 0)),
            pl.BlockSpec((1, D), lambda i: (0, 0)),
        ],
        out_specs=pl.BlockSpec((_RB, D), lambda i: (i, 0)),
        out_shape=jax.ShapeDtypeStruct((N, D), jnp.float32),
    )(acc1, g1, dinv, W1, b0)


def _k6_body(acc_ref, g2_ref, dinv_ref, batch_ref, cent_ref, b1_ref,
             wout_ref, bout_ref, out_ref, pacc):
    i = pl.program_id(0)

    @pl.when(i == 0)
    def _init():
        pacc[...] = jnp.zeros_like(pacc)

    dv = dinv_ref[...]
    a = acc_ref[0] + acc_ref[1] + g2_ref[...]
    h2 = jnp.maximum(dv * a + b1_ref[...], 0.0)

    cent = cent_ref[...]
    csq = jnp.sum(cent * cent, axis=1)
    hc = lax.dot_general(h2, cent, (((1,), (1,)), ((), ())),
                         precision=_HIGH, preferred_element_type=jnp.float32)
    sq = jnp.sum(h2 * h2, axis=1, keepdims=True) + csq[None, :] - 2.0 * hc
    dist = jnp.sqrt(jnp.maximum(sq, 1e-8))
    dist1 = jnp.concatenate(
        [dist, jnp.ones((dist.shape[0], 1), jnp.float32)], axis=1)

    gids = lax.broadcasted_iota(jnp.int32, (_RB, NUM_GRAPHS), 1)
    oh = (batch_ref[...] == gids).astype(jnp.float32)
    pacc[...] += lax.dot_general(oh, dist1, (((0,), (0,)), ((), ())),
                                 precision=_HIGH,
                                 preferred_element_type=jnp.float32)

    @pl.when(i == _GRID - 1)
    def _final():
        p = pacc[...]
        pooled = p[:, :NUM_CENTROID] / jnp.maximum(p[:, NUM_CENTROID:], 1.0)
        out_ref[...] = lax.dot_general(
            pooled, wout_ref[...], (((1,), (0,)), ((), ())),
            precision=_HIGH, preferred_element_type=jnp.float32) + bout_ref[...]


def _tc_pool_out(acc2, g2, dinv, batch2d, centroids, b1, W_out, b_out):
    return pl.pallas_call(
        _k6_body,
        grid=(_GRID,),
        in_specs=[
            pl.BlockSpec((NC, _RB, D), lambda i: (0, i, 0)),
            pl.BlockSpec((_RB, D), lambda i: (i, 0)),
            pl.BlockSpec((_RB, 1), lambda i: (i, 0)),
            pl.BlockSpec((_RB, 1), lambda i: (i, 0)),
            pl.BlockSpec((NUM_CENTROID, D), lambda i: (0, 0)),
            pl.BlockSpec((1, D), lambda i: (0, 0)),
            pl.BlockSpec((NUM_CENTROID, NUM_CLASS), lambda i: (0, 0)),
            pl.BlockSpec((1, NUM_CLASS), lambda i: (0, 0)),
        ],
        out_specs=pl.BlockSpec((NUM_GRAPHS, NUM_CLASS), lambda i: (0, 0)),
        out_shape=jax.ShapeDtypeStruct((NUM_GRAPHS, NUM_CLASS), jnp.float32),
        scratch_shapes=[pltpu.VMEM((NUM_GRAPHS, NUM_CENTROID + 1), jnp.float32)],
    )(acc2, g2, dinv, batch2d, centroids, b1, W_out, b_out)


def kernel(x, edge_index, batch, W_embed, W_gcn, b_gcn, centroids, W_out, b_out):
    src = edge_index[0]
    dst = edge_index[1]
    batch2d = batch.reshape(N, 1)
    b0 = b_gcn[0].reshape(1, D)
    b1 = b_gcn[1].reshape(1, D)
    bout = b_out.reshape(1, NUM_CLASS)

    degp = _sc_degree(dst).reshape(NC, DEG_PAD, 1)
    g1, dinv = _tc_embed_scale(x, degp, W_embed, W_gcn[0])
    acc1 = _sc_aggregate(g1, src, dst).reshape(NC, ACC_PAD, D)
    g2 = _tc_layer2_prep(acc1, g1, dinv, W_gcn[1], b0)
    acc2 = _sc_aggregate(g2, src, dst).reshape(NC, ACC_PAD, D)
    return _tc_pool_out(acc2, g2, dinv, batch2d, centroids, b1, W_out, bout)
